# Initial kernel scaffold; baseline (speedup 1.0000x reference)
import jax
import jax.numpy as jnp
from jax.experimental import pallas as pl


def kernel(box_indices, boxes):
    raise NotImplementedError("write your pallas kernel here")


# speedup vs baseline: 1.8876x; 1.8876x over previous
import functools

import jax
import jax.numpy as jnp
from jax import lax
from jax.experimental import pallas as pl
from jax.experimental.pallas import tpu as pltpu
from jax.experimental.pallas import tpu_sc as plsc

NUM_BOXES = 100000
DIM = 64
ROW = 2 * DIM
IDX_CHUNK = 128


@functools.lru_cache(maxsize=None)
def _build(batch: int):
    info = plsc.get_sparse_core_info()
    nw = info.num_cores * info.num_subcores
    b_per_w = batch // nw
    n_chunks = b_per_w // IDX_CHUNK

    mesh = plsc.VectorSubcoreMesh(core_axis_name="c", subcore_axis_name="s")

    @functools.partial(
        pl.kernel,
        out_type=jax.ShapeDtypeStruct((batch, ROW), jnp.float32),
        mesh=mesh,
        scratch_types=[
            pltpu.VMEM((n_chunks, IDX_CHUNK), jnp.int32),
            pltpu.VMEM((b_per_w, ROW), jnp.float32),
            pltpu.SemaphoreType.DMA,
        ],
    )
    def gather_kernel(idx_hbm, table_hbm, out_hbm, idx_v, rows_v, sem):
        wid = lax.axis_index("s") * info.num_cores + lax.axis_index("c")
        base = wid * b_per_w
        pltpu.sync_copy(idx_hbm.at[pl.ds(wid * n_chunks, n_chunks)], idx_v)
        copies = []
        for j in range(n_chunks):
            cp = pltpu.make_async_copy(
                table_hbm.at[idx_v.at[j]],
                rows_v.at[pl.ds(j * IDX_CHUNK, IDX_CHUNK)],
                sem,
            )
            cp.start()
            copies.append(cp)
        for cp in copies:
            cp.wait()
        pltpu.sync_copy(rows_v, out_hbm.at[pl.ds(base, b_per_w)])

    return gather_kernel


def kernel(box_indices, boxes):
    num_models, num_boxes, two, dim = boxes.shape
    batch = box_indices.shape[0]
    table = boxes.reshape(num_boxes, two * dim)
    idx2d = box_indices.astype(jnp.int32).reshape(batch // IDX_CHUNK, IDX_CHUNK)
    out = _build(batch)(idx2d, table)
    return out.reshape(num_models, batch, two, dim)


# speedup vs baseline: 2.8291x; 1.4988x over previous
import functools

import jax
import jax.numpy as jnp
from jax import lax
from jax.experimental import pallas as pl
from jax.experimental.pallas import tpu as pltpu
from jax.experimental.pallas import tpu_sc as plsc

NUM_BOXES = 100000
DIM = 64
ROWS = 2 * DIM
OUT_CHUNK = 8192


@functools.lru_cache(maxsize=None)
def _build(batch: int):
    info = plsc.get_sparse_core_info()
    nw = info.num_cores * info.num_subcores
    rows_per_w = ROWS // nw
    n_chunks = batch // OUT_CHUNK

    mesh = plsc.VectorSubcoreMesh(core_axis_name="c", subcore_axis_name="s")

    @functools.partial(
        pl.kernel,
        out_type=jax.ShapeDtypeStruct((ROWS, batch), jnp.float32),
        mesh=mesh,
        scratch_types=[
            pltpu.VMEM((batch,), jnp.int32),
            pltpu.VMEM((NUM_BOXES,), jnp.float32),
            pltpu.VMEM((OUT_CHUNK,), jnp.float32),
        ],
        compiler_params=pltpu.CompilerParams(
            use_tc_tiling_on_sc=True, needs_layout_passes=False
        ),
    )
    def gather_kernel(tbl_hbm, idx_hbm, out_hbm, idx_v, row_v, ob_v):
        wid = lax.axis_index("s") * info.num_cores + lax.axis_index("c")
        pltpu.sync_copy(idx_hbm, idx_v)
        for rl in range(rows_per_w):
            r = wid * rows_per_w + rl
            pltpu.sync_copy(tbl_hbm.at[r], row_v)
            for ch in range(n_chunks):
                def _gather(i, _ch=ch):
                    ids = idx_v[pl.ds(_ch * OUT_CHUNK + i, 16)]
                    ob_v[pl.ds(i, 16)] = plsc.load_gather(row_v, [ids])
                plsc.parallel_loop(0, OUT_CHUNK, 16, unroll=8)(_gather)
                pltpu.sync_copy(
                    ob_v, out_hbm.at[r, pl.ds(ch * OUT_CHUNK, OUT_CHUNK)]
                )

    return gather_kernel


def kernel(box_indices, boxes):
    num_models, num_boxes, two, dim = boxes.shape
    batch = box_indices.shape[0]
    tbl_t = jnp.transpose(boxes, (0, 2, 3, 1)).reshape(two * dim, num_boxes)
    out_t = _build(batch)(tbl_t, box_indices.astype(jnp.int32))
    return out_t.reshape(num_models, two, dim, batch).transpose(0, 3, 1, 2)


# speedup vs baseline: 2.9653x; 1.0481x over previous
import functools

import jax
import jax.numpy as jnp
from jax import lax
from jax.experimental import pallas as pl
from jax.experimental.pallas import tpu as pltpu
from jax.experimental.pallas import tpu_sc as plsc

NUM_BOXES = 100000
DIM = 64
ROWS = 2 * DIM
OUT_CHUNK = 4096


@functools.lru_cache(maxsize=None)
def _build(batch: int):
    info = plsc.get_sparse_core_info()
    nw = info.num_cores * info.num_subcores
    rows_per_w = ROWS // nw
    n_chunks = batch // OUT_CHUNK

    mesh = plsc.VectorSubcoreMesh(core_axis_name="c", subcore_axis_name="s")

    @functools.partial(
        pl.kernel,
        out_type=jax.ShapeDtypeStruct((ROWS, batch), jnp.float32),
        mesh=mesh,
        scratch_types=[
            pltpu.VMEM((batch,), jnp.int32),
            pltpu.VMEM((NUM_BOXES,), jnp.float32),
            pltpu.VMEM((OUT_CHUNK,), jnp.float32),
            pltpu.VMEM((OUT_CHUNK,), jnp.float32),
            pltpu.SemaphoreType.DMA,
            pltpu.SemaphoreType.DMA,
            pltpu.SemaphoreType.DMA,
        ],
        compiler_params=pltpu.CompilerParams(
            use_tc_tiling_on_sc=True, needs_layout_passes=False
        ),
    )
    def gather_kernel(
        tbl_hbm, idx_hbm, out_hbm, idx_v, row_v, ob0_v, ob1_v, sem_i, sem_r, sem_o
    ):
        obufs = (ob0_v, ob1_v)
        wid = lax.axis_index("s") * info.num_cores + lax.axis_index("c")
        idx_cp = pltpu.make_async_copy(idx_hbm, idx_v, sem_i)
        idx_cp.start()
        out_cps = []
        for rl in range(rows_per_w):
            r = wid * rows_per_w + rl
            row_cp = pltpu.make_async_copy(tbl_hbm.at[r], row_v, sem_r)
            row_cp.start()
            row_cp.wait()
            if rl == 0:
                idx_cp.wait()
            for ch in range(n_chunks):
                g = rl * n_chunks + ch
                if len(out_cps) >= 2:
                    out_cps[g - 2].wait()
                buf = obufs[g % 2]
                def _gather(i, _ch=ch, _buf=buf):
                    ids = idx_v[pl.ds(_ch * OUT_CHUNK + i, 16)]
                    _buf[pl.ds(i, 16)] = plsc.load_gather(row_v, [ids])
                plsc.parallel_loop(0, OUT_CHUNK, 16, unroll=8)(_gather)
                cp = pltpu.make_async_copy(
                    buf, out_hbm.at[r, pl.ds(ch * OUT_CHUNK, OUT_CHUNK)], sem_o
                )
                cp.start()
                out_cps.append(cp)
        for cp in out_cps[-2:]:
            cp.wait()

    return gather_kernel


def kernel(box_indices, boxes):
    num_models, num_boxes, two, dim = boxes.shape
    batch = box_indices.shape[0]
    tbl_t = jnp.transpose(boxes, (0, 2, 3, 1)).reshape(two * dim, num_boxes)
    out_t = _build(batch)(tbl_t, box_indices.astype(jnp.int32))
    return out_t.reshape(num_models, two, dim, batch).transpose(0, 3, 1, 2)


# speedup vs baseline: 2.9742x; 1.0030x over previous
import functools

import jax
import jax.numpy as jnp
from jax import lax
from jax.experimental import pallas as pl
from jax.experimental.pallas import tpu as pltpu
from jax.experimental.pallas import tpu_sc as plsc

NUM_BOXES = 100000
DIM = 64
ROWS = 2 * DIM
OUT_CHUNK = 4096


@functools.lru_cache(maxsize=None)
def _build(batch: int):
    info = plsc.get_sparse_core_info()
    nw = info.num_cores * info.num_subcores
    rows_per_w = ROWS // nw
    n_chunks = batch // OUT_CHUNK

    mesh = plsc.VectorSubcoreMesh(core_axis_name="c", subcore_axis_name="s")

    @functools.partial(
        pl.kernel,
        out_type=jax.ShapeDtypeStruct((ROWS, batch), jnp.float32),
        mesh=mesh,
        scratch_types=[
            pltpu.VMEM((batch,), jnp.int32),
            pltpu.VMEM((NUM_BOXES,), jnp.float32),
            pltpu.VMEM((OUT_CHUNK,), jnp.float32),
            pltpu.VMEM((OUT_CHUNK,), jnp.float32),
            pltpu.SemaphoreType.DMA,
            pltpu.SemaphoreType.DMA,
            pltpu.SemaphoreType.DMA,
        ],
        compiler_params=pltpu.CompilerParams(
            use_tc_tiling_on_sc=True,
            needs_layout_passes=False,
            disable_bounds_checks=True,
            disable_semaphore_checks=True,
            skip_device_barrier=True,
        ),
    )
    def gather_kernel(
        tbl_hbm, idx_hbm, out_hbm, idx_v, row_v, ob0_v, ob1_v, sem_i, sem_r, sem_o
    ):
        obufs = (ob0_v, ob1_v)
        wid = lax.axis_index("s") * info.num_cores + lax.axis_index("c")
        idx_cp = pltpu.make_async_copy(idx_hbm, idx_v, sem_i)
        idx_cp.start()
        out_cps = []
        for rl in range(rows_per_w):
            r = wid * rows_per_w + rl
            row_cp = pltpu.make_async_copy(tbl_hbm.at[r], row_v, sem_r)
            row_cp.start()
            row_cp.wait()
            if rl == 0:
                idx_cp.wait()
            for ch in range(n_chunks):
                g = rl * n_chunks + ch
                if len(out_cps) >= 2:
                    out_cps[g - 2].wait()
                buf = obufs[g % 2]
                def _gather(i, _ch=ch, _buf=buf):
                    ids = idx_v[pl.ds(_ch * OUT_CHUNK + i, 16)]
                    _buf[pl.ds(i, 16)] = plsc.load_gather(row_v, [ids])
                plsc.parallel_loop(0, OUT_CHUNK, 16, unroll=8)(_gather)
                cp = pltpu.make_async_copy(
                    buf, out_hbm.at[r, pl.ds(ch * OUT_CHUNK, OUT_CHUNK)], sem_o
                )
                cp.start()
                out_cps.append(cp)
        for cp in out_cps[-2:]:
            cp.wait()

    return gather_kernel


def kernel(box_indices, boxes):
    num_models, num_boxes, two, dim = boxes.shape
    batch = box_indices.shape[0]
    tbl_t = jnp.transpose(boxes, (0, 2, 3, 1)).reshape(two * dim, num_boxes)
    out_t = _build(batch)(tbl_t, box_indices.astype(jnp.int32))
    return out_t.reshape(num_models, two, dim, batch).transpose(0, 3, 1, 2)
